# Initial kernel scaffold; baseline (speedup 1.0000x reference)
import jax
import jax.numpy as jnp
from jax.experimental import pallas as pl


def kernel(tokens, emb_table):
    raise NotImplementedError("write your pallas kernel here")


# speedup vs baseline: 1.8694x; 1.8694x over previous
import functools

import numpy as np
import jax
import jax.numpy as jnp
from jax import lax
from jax.experimental import pallas as pl
from jax.experimental.pallas import tpu as pltpu
from jax.experimental.pallas import tpu_sc as plsc

D_MODEL = 128
MAX_LEN = 200
BATCH = 4096
N = BATCH * MAX_LEN
NUM_CORES = 2
NUM_SUBCORES = 16
NW = NUM_CORES * NUM_SUBCORES
RPW = N // NW
G = 128
NCH = RPW // G


def _pe_table():
    even = np.arange(0, D_MODEL, 2, dtype=np.float32)
    inv = np.reciprocal(
        np.power(np.float32(10000.0), even / np.float32(D_MODEL))
    ).astype(np.float32)
    pos = np.arange(MAX_LEN, dtype=np.float32).reshape(MAX_LEN, 1)
    ang = (pos * inv.reshape(1, D_MODEL // 2)).astype(np.float32)
    pe = np.empty((MAX_LEN, D_MODEL), dtype=np.float32)
    pe[:, 0::2] = np.sin(ang)
    pe[:, 1::2] = np.cos(ang)
    return jnp.asarray(pe)


_mesh = plsc.VectorSubcoreMesh(core_axis_name="c", subcore_axis_name="s")


@functools.partial(
    pl.kernel,
    mesh=_mesh,
    out_type=jax.ShapeDtypeStruct((N, D_MODEL), jnp.float32),
    scratch_types=[
        pltpu.VMEM((G,), jnp.int32),
        pltpu.VMEM((G, D_MODEL), jnp.float32),
        pltpu.VMEM((MAX_LEN, D_MODEL), jnp.float32),
        pltpu.SemaphoreType.DMA,
    ],
)
def _emb_kernel(tokens_hbm, table_hbm, pe_hbm, out_hbm, idx_v, rows_v, pe_v, gsem):
    wid = lax.axis_index("s") * NUM_CORES + lax.axis_index("c")
    wbase = wid * RPW
    pltpu.sync_copy(pe_hbm, pe_v)

    def chunk(c, carry):
        base = wbase + c * G
        pltpu.sync_copy(tokens_hbm.at[pl.ds(base, G)], idx_v)
        pltpu.async_copy(table_hbm.at[idx_v], rows_v, gsem).wait()
        phase = lax.rem(c * G, MAX_LEN)

        def row(r, rcarry):
            pr = lax.rem(phase + r, MAX_LEN)
            for g in range(D_MODEL // 16):
                s = pl.ds(g * 16, 16)
                rows_v[r, s] = rows_v[r, s] + pe_v[pr, s]
            return rcarry

        lax.fori_loop(0, G, row, 0)
        pltpu.sync_copy(rows_v, out_hbm.at[pl.ds(base, G)])
        return carry

    lax.fori_loop(0, NCH, chunk, 0)


def kernel(tokens, emb_table):
    pe = _pe_table()
    out = _emb_kernel(tokens.reshape(N), emb_table, pe)
    return out.reshape(BATCH, MAX_LEN, D_MODEL)


# speedup vs baseline: 7.5052x; 4.0147x over previous
import functools

import numpy as np
import jax
import jax.numpy as jnp
from jax import lax
from jax.experimental import pallas as pl
from jax.experimental.pallas import tpu as pltpu
from jax.experimental.pallas import tpu_sc as plsc

D_MODEL = 128
MAX_LEN = 200
BATCH = 4096
N = BATCH * MAX_LEN
NUM_CORES = 2
NUM_SUBCORES = 16
NW = NUM_CORES * NUM_SUBCORES
RPW = N // NW
CH = MAX_LEN
NCH = RPW // CH
G0 = 128
G1 = CH - G0


def _pe_table():
    even = np.arange(0, D_MODEL, 2, dtype=np.float32)
    inv = np.reciprocal(
        np.power(np.float32(10000.0), even / np.float32(D_MODEL))
    ).astype(np.float32)
    pos = np.arange(MAX_LEN, dtype=np.float32).reshape(MAX_LEN, 1)
    ang = (pos * inv.reshape(1, D_MODEL // 2)).astype(np.float32)
    pe = np.empty((MAX_LEN, D_MODEL), dtype=np.float32)
    pe[:, 0::2] = np.sin(ang)
    pe[:, 1::2] = np.cos(ang)
    return jnp.asarray(pe)


_mesh = plsc.VectorSubcoreMesh(core_axis_name="c", subcore_axis_name="s")


@functools.partial(
    pl.kernel,
    mesh=_mesh,
    out_type=jax.ShapeDtypeStruct((N, D_MODEL), jnp.float32),
    scratch_types=[
        pltpu.VMEM((RPW,), jnp.int32),
        pltpu.VMEM((2, CH, D_MODEL), jnp.float32),
        pltpu.VMEM((MAX_LEN, D_MODEL), jnp.float32),
        pltpu.SemaphoreType.DMA,
        pltpu.SemaphoreType.DMA,
        pltpu.SemaphoreType.DMA,
        pltpu.SemaphoreType.DMA,
    ],
)
def _emb_kernel(tokens_hbm, table_hbm, pe_hbm, out_hbm,
                idx_v, rows_v, pe_v, g0sem, g1sem, s0sem, s1sem):
    wid = lax.axis_index("s") * NUM_CORES + lax.axis_index("c")
    wbase = wid * RPW
    pltpu.sync_copy(tokens_hbm.at[pl.ds(wbase, RPW)], idx_v)
    pltpu.sync_copy(pe_hbm, pe_v)

    gsems = (g0sem, g1sem)
    ssems = (s0sem, s1sem)

    def start_gather(c, b):
        off = c * CH
        pltpu.async_copy(table_hbm.at[idx_v.at[pl.ds(off, G0)]],
                         rows_v.at[b, pl.ds(0, G0)], gsems[b])
        pltpu.async_copy(table_hbm.at[idx_v.at[pl.ds(off + G0, G1)]],
                         rows_v.at[b, pl.ds(G0, G1)], gsems[b])

    def wait_gather(c, b):
        off = c * CH
        pltpu.make_async_copy(table_hbm.at[idx_v.at[pl.ds(off, G0)]],
                              rows_v.at[b, pl.ds(0, G0)], gsems[b]).wait()
        pltpu.make_async_copy(table_hbm.at[idx_v.at[pl.ds(off + G0, G1)]],
                              rows_v.at[b, pl.ds(G0, G1)], gsems[b]).wait()

    def store_chunk(c, b):
        pltpu.async_copy(rows_v.at[b], out_hbm.at[pl.ds(wbase + c * CH, CH)],
                         ssems[b])

    def wait_store(c, b):
        pltpu.make_async_copy(rows_v.at[b],
                              out_hbm.at[pl.ds(wbase + c * CH, CH)],
                              ssems[b]).wait()

    start_gather(0, 0)

    def body(c2, carry):
        for b in (0, 1):
            c = c2 * 2 + b
            other = 1 - b
            if b == 0:
                @pl.when(c >= 1)
                def _():
                    wait_store(c - 1, other)

                start_gather(c + 1, other)
            else:
                wait_store(c - 1, other)

                @pl.when(c + 1 < NCH)
                def _():
                    start_gather(c + 1, other)

            wait_gather(c, b)

            def add_pe(r2, rcarry):
                for u in (0, 1):
                    r = r2 * 2 + u
                    for g in range(D_MODEL // 16):
                        s = pl.ds(g * 16, 16)
                        plsc.addupdate(rows_v.at[b, r, s], pe_v[r, s])
                return rcarry

            lax.fori_loop(0, CH // 2, add_pe, 0)
            store_chunk(c, b)
        return carry

    lax.fori_loop(0, NCH // 2, body, 0)
    wait_store(NCH - 1, 1)


def kernel(tokens, emb_table):
    pe = _pe_table()
    out = _emb_kernel(tokens.reshape(N), emb_table, pe)
    return out.reshape(BATCH, MAX_LEN, D_MODEL)


# speedup vs baseline: 9.0313x; 1.2033x over previous
import functools

import numpy as np
import jax
import jax.numpy as jnp
from jax import lax
from jax.experimental import pallas as pl
from jax.experimental.pallas import tpu as pltpu
from jax.experimental.pallas import tpu_sc as plsc

D_MODEL = 128
MAX_LEN = 200
BATCH = 4096
N = BATCH * MAX_LEN
NUM_CORES = 2
NUM_SUBCORES = 16
NW = NUM_CORES * NUM_SUBCORES
RPW = N // NW
CH = MAX_LEN
NCH = RPW // CH
G0 = 128
G1 = CH - G0


def _pe_table():
    even = np.arange(0, D_MODEL, 2, dtype=np.float32)
    inv = np.reciprocal(
        np.power(np.float32(10000.0), even / np.float32(D_MODEL))
    ).astype(np.float32)
    pos = np.arange(MAX_LEN, dtype=np.float32).reshape(MAX_LEN, 1)
    ang = (pos * inv.reshape(1, D_MODEL // 2)).astype(np.float32)
    pe = np.empty((MAX_LEN, D_MODEL), dtype=np.float32)
    pe[:, 0::2] = np.sin(ang)
    pe[:, 1::2] = np.cos(ang)
    return jnp.asarray(pe)


_mesh = plsc.VectorSubcoreMesh(core_axis_name="c", subcore_axis_name="s")


@functools.partial(
    pl.kernel,
    mesh=_mesh,
    out_type=jax.ShapeDtypeStruct((N, D_MODEL), jnp.float32),
    scratch_types=[
        pltpu.VMEM((RPW,), jnp.int32),
        pltpu.VMEM((2, CH, D_MODEL), jnp.float32),
        pltpu.VMEM((MAX_LEN, D_MODEL), jnp.float32),
        pltpu.SemaphoreType.DMA,
        pltpu.SemaphoreType.DMA,
        pltpu.SemaphoreType.DMA,
        pltpu.SemaphoreType.DMA,
    ],
)
def _emb_kernel(tokens_hbm, table_hbm, pe_hbm, out_hbm,
                idx_v, rows_v, pe_v, g0sem, g1sem, s0sem, s1sem):
    wid = lax.axis_index("s") * NUM_CORES + lax.axis_index("c")
    wbase = wid * RPW
    pltpu.sync_copy(tokens_hbm.at[pl.ds(wbase, RPW)], idx_v)
    pltpu.sync_copy(pe_hbm, pe_v)

    gsems = (g0sem, g1sem)
    ssems = (s0sem, s1sem)

    def start_gather(c, b):
        off = c * CH
        pltpu.async_copy(table_hbm.at[idx_v.at[pl.ds(off, G0)]],
                         rows_v.at[b, pl.ds(0, G0)], gsems[b])
        pltpu.async_copy(table_hbm.at[idx_v.at[pl.ds(off + G0, G1)]],
                         rows_v.at[b, pl.ds(G0, G1)], gsems[b])

    def wait_gather(c, b):
        off = c * CH
        pltpu.make_async_copy(table_hbm.at[idx_v.at[pl.ds(off, G0)]],
                              rows_v.at[b, pl.ds(0, G0)], gsems[b]).wait()
        pltpu.make_async_copy(table_hbm.at[idx_v.at[pl.ds(off + G0, G1)]],
                              rows_v.at[b, pl.ds(G0, G1)], gsems[b]).wait()

    def store_chunk(c, b):
        pltpu.async_copy(rows_v.at[b], out_hbm.at[pl.ds(wbase + c * CH, CH)],
                         ssems[b])

    def wait_store(c, b):
        pltpu.make_async_copy(rows_v.at[b],
                              out_hbm.at[pl.ds(wbase + c * CH, CH)],
                              ssems[b]).wait()

    start_gather(0, 0)

    def body(c2, carry):
        for b in (0, 1):
            c = c2 * 2 + b
            other = 1 - b
            if b == 0:
                @pl.when(c >= 1)
                def _():
                    wait_store(c - 1, other)

                start_gather(c + 1, other)
            else:
                wait_store(c - 1, other)

                @pl.when(c + 1 < NCH)
                def _():
                    start_gather(c + 1, other)

            wait_gather(c, b)

            @functools.partial(plsc.parallel_loop, 0, CH, unroll=4)
            def add_pe(r):
                for g in range(D_MODEL // 16):
                    s = pl.ds(g * 16, 16)
                    plsc.addupdate(rows_v.at[b, r, s], pe_v[r, s])
            store_chunk(c, b)
        return carry

    lax.fori_loop(0, NCH // 2, body, 0)
    wait_store(NCH - 1, 1)


def kernel(tokens, emb_table):
    pe = _pe_table()
    out = _emb_kernel(tokens.reshape(N), emb_table, pe)
    return out.reshape(BATCH, MAX_LEN, D_MODEL)
